# Initial kernel scaffold; baseline (speedup 1.0000x reference)
import jax
import jax.numpy as jnp
from jax.experimental import pallas as pl


def kernel(x, edge_index, W1, b1, W2, b2, W3, b3):
    raise NotImplementedError("write your pallas kernel here")


# speedup vs baseline: 9.8493x; 9.8493x over previous
import functools

import jax
import jax.numpy as jnp
from jax import lax
from jax.experimental import pallas as pl
from jax.experimental.pallas import tpu as pltpu
from jax.experimental.pallas import tpu_sc as plsc

N = 10000
E = 160000
D = 256
DH = 128
NC = 2
NS = 16
CHUNK = 125
ROWS = E // CHUNK
TROWS = ROWS // NS
DROWS = ROWS // (NC * NS)
S0 = 632

_mesh = plsc.VectorSubcoreMesh(core_axis_name="c", subcore_axis_name="s")


def _stripe(s):
    return jnp.where(s < NS - 1, s * S0, N - S0)


@functools.partial(
    pl.kernel,
    out_type=jax.ShapeDtypeStruct((NC * N, DH), jnp.float32),
    mesh=_mesh,
    scratch_types=[
        pltpu.VMEM((TROWS, CHUNK), jnp.int32),
        pltpu.VMEM((TROWS, CHUNK), jnp.int32),
        pltpu.VMEM((CHUNK, DH), jnp.float32),
        pltpu.VMEM_SHARED((N, DH), jnp.float32),
        pltpu.SemaphoreType.DMA,
    ],
)
def _edge_agg(g2, src3, dst2, out, srcv, dstv, buf0, acc, sem0):
    c = lax.axis_index("c")
    s = lax.axis_index("s")
    o = _stripe(s)
    pltpu.sync_copy(src3.at[pl.ds(c * ROWS + s * TROWS, TROWS)], srcv)
    pltpu.sync_copy(dst2.at[pl.ds(s * TROWS, TROWS)], dstv)
    pltpu.sync_copy(g2.at[pl.ds(c * N + o, S0)], acc.at[pl.ds(o, S0)])
    plsc.subcore_barrier()

    def body(j, _):
        pltpu.async_copy(g2.at[srcv.at[j]], buf0, sem0).wait()
        pltpu.sync_copy(buf0, acc.at[dstv.at[j]], add=True)
        return 0

    lax.fori_loop(0, TROWS, body, 0)
    plsc.subcore_barrier()
    pltpu.sync_copy(acc.at[pl.ds(o, S0)], out.at[pl.ds(c * N + o, S0)])


BLK = 1000


def _tc1_body(x_ref, w_ref, deg_ref, g_ref, dinv_ref):
    dinv = lax.rsqrt(deg_ref[:, 0:1])
    g = jnp.dot(x_ref[...], w_ref[...], preferred_element_type=jnp.float32) * dinv
    g_ref[0] = g[:, :DH]
    g_ref[1] = g[:, DH:]
    dinv_ref[...] = dinv


_tc1 = pl.pallas_call(
    _tc1_body,
    grid=(N // BLK,),
    in_specs=[
        pl.BlockSpec((BLK, D), lambda i: (i, 0)),
        pl.BlockSpec((D, D), lambda i: (0, 0)),
        pl.BlockSpec((BLK, DH), lambda i: (i, 0)),
    ],
    out_specs=[
        pl.BlockSpec((2, BLK, DH), lambda i: (0, i, 0)),
        pl.BlockSpec((BLK, 1), lambda i: (i, 0)),
    ],
    out_shape=[
        jax.ShapeDtypeStruct((2, N, DH), jnp.float32),
        jax.ShapeDtypeStruct((N, 1), jnp.float32),
    ],
)


def _tc_mid_body(sa_ref, sb_ref, dinv_ref, b_ref, w_ref, g_ref):
    sfull = jnp.concatenate([sa_ref[0], sb_ref[0]], axis=1)
    h = jnp.maximum(sfull * dinv_ref[...] + b_ref[...], 0.0)
    g = jnp.dot(h, w_ref[...], preferred_element_type=jnp.float32) * dinv_ref[...]
    g_ref[0] = g[:, :DH]
    g_ref[1] = g[:, DH:]


_tc_mid = pl.pallas_call(
    _tc_mid_body,
    grid=(N // BLK,),
    in_specs=[
        pl.BlockSpec((1, BLK, DH), lambda i: (0, i, 0)),
        pl.BlockSpec((1, BLK, DH), lambda i: (1, i, 0)),
        pl.BlockSpec((BLK, 1), lambda i: (i, 0)),
        pl.BlockSpec((1, D), lambda i: (0, 0)),
        pl.BlockSpec((D, D), lambda i: (0, 0)),
    ],
    out_specs=pl.BlockSpec((2, BLK, DH), lambda i: (0, i, 0)),
    out_shape=jax.ShapeDtypeStruct((2, N, DH), jnp.float32),
)


def _tc_fin_body(sa_ref, sb_ref, dinv_ref, b_ref, out_ref):
    sfull = jnp.concatenate([sa_ref[0], sb_ref[0]], axis=1)
    out_ref[...] = sfull * dinv_ref[...] + b_ref[...]


_tc_fin = pl.pallas_call(
    _tc_fin_body,
    grid=(N // BLK,),
    in_specs=[
        pl.BlockSpec((1, BLK, DH), lambda i: (0, i, 0)),
        pl.BlockSpec((1, BLK, DH), lambda i: (1, i, 0)),
        pl.BlockSpec((BLK, 1), lambda i: (i, 0)),
        pl.BlockSpec((1, D), lambda i: (0, 0)),
    ],
    out_specs=pl.BlockSpec((BLK, D), lambda i: (i, 0)),
    out_shape=jax.ShapeDtypeStruct((N, D), jnp.float32),
)


def kernel(x, edge_index, W1, b1, W2, b2, W3, b3):
    src = edge_index[0]
    dst = edge_index[1]
    src3 = jnp.concatenate([src, src + N]).reshape(NC * ROWS, CHUNK)
    dst2 = dst.reshape(ROWS, CHUNK)
    degc = _edge_agg(jnp.ones((NC * N, DH), jnp.float32), src3, dst2)
    g, dinv = _tc1(x, W1, degc)
    s3 = _edge_agg(g.reshape(NC * N, DH), src3, dst2).reshape(NC, N, DH)
    g = _tc_mid(s3, s3, dinv, b1.reshape(1, D), W2)
    s3 = _edge_agg(g.reshape(NC * N, DH), src3, dst2).reshape(NC, N, DH)
    g = _tc_mid(s3, s3, dinv, b2.reshape(1, D), W3)
    s3 = _edge_agg(g.reshape(NC * N, DH), src3, dst2).reshape(NC, N, DH)
    return _tc_fin(s3, s3, dinv, b3.reshape(1, D))


# speedup vs baseline: 15.6537x; 1.5893x over previous
import functools

import jax
import jax.numpy as jnp
from jax import lax
from jax.experimental import pallas as pl
from jax.experimental.pallas import tpu as pltpu
from jax.experimental.pallas import tpu_sc as plsc

N = 10000
E = 160000
D = 256
DH = 128
NC = 2
NS = 16
CHUNK = 125
ROWS = E // CHUNK
TROWS = ROWS // NS
DROWS = ROWS // (NC * NS)
S0 = 632

_mesh = plsc.VectorSubcoreMesh(core_axis_name="c", subcore_axis_name="s")


def _stripe(s):
    return jnp.where(s < NS - 1, s * S0, N - S0)


EC = 50
EROWS = E // EC
ETROWS = EROWS // NS
NBUF = 4
PHASES = 5
PROWS = ETROWS // PHASES
PGROUPS = PROWS // NBUF


@functools.partial(
    pl.kernel,
    out_type=jax.ShapeDtypeStruct((NC * N, DH), jnp.float32),
    mesh=_mesh,
    scratch_types=[
        pltpu.VMEM((PROWS, EC), jnp.int32),
        pltpu.VMEM((PROWS, EC), jnp.int32),
        [pltpu.VMEM((EC, DH), jnp.float32) for _ in range(NBUF)],
        pltpu.VMEM_SHARED((N, DH), jnp.float32),
        pltpu.SemaphoreType.DMA,
        pltpu.SemaphoreType.DMA,
    ],
)
def _edge_agg(g2, src3, dst2, out, srcv, dstv, bufs, acc, sem_g, sem_s):
    c = lax.axis_index("c")
    s = lax.axis_index("s")
    o = _stripe(s)
    pltpu.sync_copy(g2.at[pl.ds(c * N + o, S0)], acc.at[pl.ds(o, S0)])
    plsc.subcore_barrier()

    for p in range(PHASES):
        pltpu.sync_copy(
            src3.at[pl.ds(c * EROWS + s * ETROWS + p * PROWS, PROWS)], srcv)
        pltpu.sync_copy(dst2.at[pl.ds(s * ETROWS + p * PROWS, PROWS)], dstv)
        for k in range(NBUF):
            pltpu.async_copy(g2.at[srcv.at[k]], bufs[k], sem_g)

        def body(i, _):
            base = i * NBUF
            for k in range(NBUF):
                pltpu.make_async_copy(
                    g2.at[srcv.at[base + k]], bufs[k], sem_g).wait()
                pltpu.async_copy(bufs[k], acc.at[dstv.at[base + k]], sem_s,
                                 add=True)
            for k in range(NBUF):
                jn = jnp.minimum(base + NBUF + k, PROWS - 1)
                pltpu.make_async_copy(bufs[k], acc.at[dstv.at[0]], sem_s).wait()

                @pl.when(i < PGROUPS - 1)
                def _():
                    pltpu.async_copy(g2.at[srcv.at[jn]], bufs[k], sem_g)

            return 0

        lax.fori_loop(0, PGROUPS, body, 0)

    plsc.subcore_barrier()
    pltpu.sync_copy(acc.at[pl.ds(o, S0)], out.at[pl.ds(c * N + o, S0)])


@functools.partial(
    pl.kernel,
    out_type=jax.ShapeDtypeStruct((NC * N, DH), jnp.float32),
    mesh=_mesh,
    scratch_types=[
        pltpu.VMEM((DROWS, CHUNK), jnp.int32),
        pltpu.VMEM((CHUNK, DH), jnp.float32),
        pltpu.VMEM_SHARED((N, DH), jnp.float32),
    ],
)
def _cnt_sc(init2, dst2, out, dstv, ones_v, acc):
    c = lax.axis_index("c")
    s = lax.axis_index("s")
    t = c * NS + s
    o = _stripe(s)
    pltpu.sync_copy(dst2.at[pl.ds(t * DROWS, DROWS)], dstv)
    pltpu.sync_copy(init2.at[pl.ds(c * N + o, S0)], acc.at[pl.ds(o, S0)])

    def fill(i, _):
        for k in range(DH // 16):
            ones_v[i, pl.ds(k * 16, 16)] = jnp.full((16,), 1.0, jnp.float32)
        return 0

    lax.fori_loop(0, CHUNK, fill, 0)
    plsc.subcore_barrier()

    def body(j, _):
        pltpu.sync_copy(ones_v, acc.at[dstv.at[j]], add=True)
        return 0

    lax.fori_loop(0, DROWS, body, 0)
    plsc.subcore_barrier()
    pltpu.sync_copy(acc.at[pl.ds(o, S0)], out.at[pl.ds(c * N + o, S0)])


BLK = 1000


def _tc1_body(x_ref, w_ref, da_ref, db_ref, g_ref, dinv_ref):
    dinv = lax.rsqrt(da_ref[:, 0:1] + db_ref[:, 0:1])
    g = jnp.dot(x_ref[...], w_ref[...], preferred_element_type=jnp.float32) * dinv
    g_ref[0] = g[:, :DH]
    g_ref[1] = g[:, DH:]
    dinv_ref[...] = dinv


_tc1 = pl.pallas_call(
    _tc1_body,
    grid=(N // BLK,),
    in_specs=[
        pl.BlockSpec((BLK, D), lambda i: (i, 0)),
        pl.BlockSpec((D, D), lambda i: (0, 0)),
        pl.BlockSpec((BLK, DH), lambda i: (i, 0)),
        pl.BlockSpec((BLK, DH), lambda i: (i + N // BLK, 0)),
    ],
    out_specs=[
        pl.BlockSpec((2, BLK, DH), lambda i: (0, i, 0)),
        pl.BlockSpec((BLK, 1), lambda i: (i, 0)),
    ],
    out_shape=[
        jax.ShapeDtypeStruct((2, N, DH), jnp.float32),
        jax.ShapeDtypeStruct((N, 1), jnp.float32),
    ],
)


def _tc_mid_body(sa_ref, sb_ref, dinv_ref, b_ref, w_ref, g_ref):
    sfull = jnp.concatenate([sa_ref[0], sb_ref[0]], axis=1)
    h = jnp.maximum(sfull * dinv_ref[...] + b_ref[...], 0.0)
    g = jnp.dot(h, w_ref[...], preferred_element_type=jnp.float32) * dinv_ref[...]
    g_ref[0] = g[:, :DH]
    g_ref[1] = g[:, DH:]


_tc_mid = pl.pallas_call(
    _tc_mid_body,
    grid=(N // BLK,),
    in_specs=[
        pl.BlockSpec((1, BLK, DH), lambda i: (0, i, 0)),
        pl.BlockSpec((1, BLK, DH), lambda i: (1, i, 0)),
        pl.BlockSpec((BLK, 1), lambda i: (i, 0)),
        pl.BlockSpec((1, D), lambda i: (0, 0)),
        pl.BlockSpec((D, D), lambda i: (0, 0)),
    ],
    out_specs=pl.BlockSpec((2, BLK, DH), lambda i: (0, i, 0)),
    out_shape=jax.ShapeDtypeStruct((2, N, DH), jnp.float32),
)


def _tc_fin_body(sa_ref, sb_ref, dinv_ref, b_ref, out_ref):
    sfull = jnp.concatenate([sa_ref[0], sb_ref[0]], axis=1)
    out_ref[...] = sfull * dinv_ref[...] + b_ref[...]


_tc_fin = pl.pallas_call(
    _tc_fin_body,
    grid=(N // BLK,),
    in_specs=[
        pl.BlockSpec((1, BLK, DH), lambda i: (0, i, 0)),
        pl.BlockSpec((1, BLK, DH), lambda i: (1, i, 0)),
        pl.BlockSpec((BLK, 1), lambda i: (i, 0)),
        pl.BlockSpec((1, D), lambda i: (0, 0)),
    ],
    out_specs=pl.BlockSpec((BLK, D), lambda i: (i, 0)),
    out_shape=jax.ShapeDtypeStruct((N, D), jnp.float32),
)


def kernel(x, edge_index, W1, b1, W2, b2, W3, b3):
    src = edge_index[0]
    dst = edge_index[1]
    src3 = jnp.concatenate([src, src + N]).reshape(NC * EROWS, EC)
    dst2 = dst.reshape(EROWS, EC)
    dst2c = dst.reshape(ROWS, CHUNK)
    init2 = jnp.concatenate([jnp.ones((N, DH), jnp.float32),
                             jnp.zeros((N, DH), jnp.float32)])
    degc = _cnt_sc(init2, dst2c)
    g, dinv = _tc1(x, W1, degc, degc)
    s3 = _edge_agg(g.reshape(NC * N, DH), src3, dst2).reshape(NC, N, DH)
    g = _tc_mid(s3, s3, dinv, b1.reshape(1, D), W2)
    s3 = _edge_agg(g.reshape(NC * N, DH), src3, dst2).reshape(NC, N, DH)
    g = _tc_mid(s3, s3, dinv, b2.reshape(1, D), W3)
    s3 = _edge_agg(g.reshape(NC * N, DH), src3, dst2).reshape(NC, N, DH)
    return _tc_fin(s3, s3, dinv, b3.reshape(1, D))


# speedup vs baseline: 15.8026x; 1.0095x over previous
import functools

import jax
import jax.numpy as jnp
from jax import lax
from jax.experimental import pallas as pl
from jax.experimental.pallas import tpu as pltpu
from jax.experimental.pallas import tpu_sc as plsc

N = 10000
E = 160000
D = 256
DH = 128
NC = 2
NS = 16
CHUNK = 125
ROWS = E // CHUNK
TROWS = ROWS // NS
DROWS = ROWS // (NC * NS)
S0 = 632

_mesh = plsc.VectorSubcoreMesh(core_axis_name="c", subcore_axis_name="s")


def _stripe(s):
    return jnp.where(s < NS - 1, s * S0, N - S0)


EC = 125
EROWS = E // EC
ETROWS = EROWS // NS
NBUF = 2
PHASES = 5
PROWS = ETROWS // PHASES
PGROUPS = PROWS // NBUF


@functools.partial(
    pl.kernel,
    out_type=jax.ShapeDtypeStruct((NC * N, DH), jnp.float32),
    mesh=_mesh,
    scratch_types=[
        pltpu.VMEM((PROWS, EC), jnp.int32),
        pltpu.VMEM((PROWS, EC), jnp.int32),
        [pltpu.VMEM((EC, DH), jnp.float32) for _ in range(NBUF)],
        pltpu.VMEM_SHARED((N, DH), jnp.float32),
        pltpu.SemaphoreType.DMA,
        pltpu.SemaphoreType.DMA,
    ],
)
def _edge_agg(g2, src3, dst2, out, srcv, dstv, bufs, acc, sem_g, sem_s):
    c = lax.axis_index("c")
    s = lax.axis_index("s")
    o = _stripe(s)
    pltpu.sync_copy(g2.at[pl.ds(c * N + o, S0)], acc.at[pl.ds(o, S0)])
    plsc.subcore_barrier()

    for p in range(PHASES):
        pltpu.sync_copy(
            src3.at[pl.ds(c * EROWS + s * ETROWS + p * PROWS, PROWS)], srcv)
        pltpu.sync_copy(dst2.at[pl.ds(s * ETROWS + p * PROWS, PROWS)], dstv)
        for k in range(NBUF):
            pltpu.async_copy(g2.at[srcv.at[k]], bufs[k], sem_g)

        def body(i, _):
            base = i * NBUF
            for k in range(NBUF):
                pltpu.make_async_copy(
                    g2.at[srcv.at[base + k]], bufs[k], sem_g).wait()
                pltpu.async_copy(bufs[k], acc.at[dstv.at[base + k]], sem_s,
                                 add=True)
            for k in range(NBUF):
                jn = jnp.minimum(base + NBUF + k, PROWS - 1)
                pltpu.make_async_copy(bufs[k], acc.at[dstv.at[0]], sem_s).wait()

                @pl.when(i < PGROUPS - 1)
                def _():
                    pltpu.async_copy(g2.at[srcv.at[jn]], bufs[k], sem_g)

            return 0

        lax.fori_loop(0, PGROUPS, body, 0)

    plsc.subcore_barrier()
    pltpu.sync_copy(acc.at[pl.ds(o, S0)], out.at[pl.ds(c * N + o, S0)])


@functools.partial(
    pl.kernel,
    out_type=jax.ShapeDtypeStruct((NC * N, DH), jnp.float32),
    mesh=_mesh,
    scratch_types=[
        pltpu.VMEM((DROWS, CHUNK), jnp.int32),
        pltpu.VMEM((CHUNK, DH), jnp.float32),
        pltpu.VMEM_SHARED((N, DH), jnp.float32),
    ],
)
def _cnt_sc(init2, dst2, out, dstv, ones_v, acc):
    c = lax.axis_index("c")
    s = lax.axis_index("s")
    t = c * NS + s
    o = _stripe(s)
    pltpu.sync_copy(dst2.at[pl.ds(t * DROWS, DROWS)], dstv)
    pltpu.sync_copy(init2.at[pl.ds(c * N + o, S0)], acc.at[pl.ds(o, S0)])

    def fill(i, _):
        for k in range(DH // 16):
            ones_v[i, pl.ds(k * 16, 16)] = jnp.full((16,), 1.0, jnp.float32)
        return 0

    lax.fori_loop(0, CHUNK, fill, 0)
    plsc.subcore_barrier()

    def body(j, _):
        pltpu.sync_copy(ones_v, acc.at[dstv.at[j]], add=True)
        return 0

    lax.fori_loop(0, DROWS, body, 0)
    plsc.subcore_barrier()
    pltpu.sync_copy(acc.at[pl.ds(o, S0)], out.at[pl.ds(c * N + o, S0)])


BLK = 1000


def _tc1_body(x_ref, w_ref, da_ref, db_ref, g_ref, dinv_ref):
    dinv = lax.rsqrt(da_ref[:, 0:1] + db_ref[:, 0:1])
    g = jnp.dot(x_ref[...], w_ref[...], preferred_element_type=jnp.float32) * dinv
    g_ref[0] = g[:, :DH]
    g_ref[1] = g[:, DH:]
    dinv_ref[...] = dinv


_tc1 = pl.pallas_call(
    _tc1_body,
    grid=(N // BLK,),
    in_specs=[
        pl.BlockSpec((BLK, D), lambda i: (i, 0)),
        pl.BlockSpec((D, D), lambda i: (0, 0)),
        pl.BlockSpec((BLK, DH), lambda i: (i, 0)),
        pl.BlockSpec((BLK, DH), lambda i: (i + N // BLK, 0)),
    ],
    out_specs=[
        pl.BlockSpec((2, BLK, DH), lambda i: (0, i, 0)),
        pl.BlockSpec((BLK, 1), lambda i: (i, 0)),
    ],
    out_shape=[
        jax.ShapeDtypeStruct((2, N, DH), jnp.float32),
        jax.ShapeDtypeStruct((N, 1), jnp.float32),
    ],
)


def _tc_mid_body(sa_ref, sb_ref, dinv_ref, b_ref, w_ref, g_ref):
    sfull = jnp.concatenate([sa_ref[0], sb_ref[0]], axis=1)
    h = jnp.maximum(sfull * dinv_ref[...] + b_ref[...], 0.0)
    g = jnp.dot(h, w_ref[...], preferred_element_type=jnp.float32) * dinv_ref[...]
    g_ref[0] = g[:, :DH]
    g_ref[1] = g[:, DH:]


_tc_mid = pl.pallas_call(
    _tc_mid_body,
    grid=(N // BLK,),
    in_specs=[
        pl.BlockSpec((1, BLK, DH), lambda i: (0, i, 0)),
        pl.BlockSpec((1, BLK, DH), lambda i: (1, i, 0)),
        pl.BlockSpec((BLK, 1), lambda i: (i, 0)),
        pl.BlockSpec((1, D), lambda i: (0, 0)),
        pl.BlockSpec((D, D), lambda i: (0, 0)),
    ],
    out_specs=pl.BlockSpec((2, BLK, DH), lambda i: (0, i, 0)),
    out_shape=jax.ShapeDtypeStruct((2, N, DH), jnp.float32),
)


def _tc_fin_body(sa_ref, sb_ref, dinv_ref, b_ref, out_ref):
    sfull = jnp.concatenate([sa_ref[0], sb_ref[0]], axis=1)
    out_ref[...] = sfull * dinv_ref[...] + b_ref[...]


_tc_fin = pl.pallas_call(
    _tc_fin_body,
    grid=(N // BLK,),
    in_specs=[
        pl.BlockSpec((1, BLK, DH), lambda i: (0, i, 0)),
        pl.BlockSpec((1, BLK, DH), lambda i: (1, i, 0)),
        pl.BlockSpec((BLK, 1), lambda i: (i, 0)),
        pl.BlockSpec((1, D), lambda i: (0, 0)),
    ],
    out_specs=pl.BlockSpec((BLK, D), lambda i: (i, 0)),
    out_shape=jax.ShapeDtypeStruct((N, D), jnp.float32),
)


def kernel(x, edge_index, W1, b1, W2, b2, W3, b3):
    src = edge_index[0]
    dst = edge_index[1]
    src3 = jnp.concatenate([src, src + N]).reshape(NC * EROWS, EC)
    dst2 = dst.reshape(EROWS, EC)
    dst2c = dst.reshape(ROWS, CHUNK)
    init2 = jnp.concatenate([jnp.ones((N, DH), jnp.float32),
                             jnp.zeros((N, DH), jnp.float32)])
    degc = _cnt_sc(init2, dst2c)
    g, dinv = _tc1(x, W1, degc, degc)
    s3 = _edge_agg(g.reshape(NC * N, DH), src3, dst2).reshape(NC, N, DH)
    g = _tc_mid(s3, s3, dinv, b1.reshape(1, D), W2)
    s3 = _edge_agg(g.reshape(NC * N, DH), src3, dst2).reshape(NC, N, DH)
    g = _tc_mid(s3, s3, dinv, b2.reshape(1, D), W3)
    s3 = _edge_agg(g.reshape(NC * N, DH), src3, dst2).reshape(NC, N, DH)
    return _tc_fin(s3, s3, dinv, b3.reshape(1, D))
